# Initial kernel scaffold; baseline (speedup 1.0000x reference)
import jax
import jax.numpy as jnp
from jax.experimental import pallas as pl


def kernel(x, params, edge_index):
    raise NotImplementedError("write your pallas kernel here")


# speedup vs baseline: 1.6689x; 1.6689x over previous
import functools

import jax
import jax.numpy as jnp
from jax import lax
from jax.experimental import pallas as pl
from jax.experimental.pallas import tpu as pltpu
from jax.experimental.pallas import tpu_sc as plsc

N = 10000
E = 320000
D = 128
H = 64

NC = 2
NS = 16
NW = NC * NS
CH = 128
NCHUNK = 80
PERW = CH * NCHUNK
EPAD = PERW * NW
WIDTH = 2 * D
NACC = 10112
RPT = NACC // NS


_NB = 1000


def _node_kernel(h_ref, pw1_ref, pb1_ref, pw2_ref, pb2_ref,
                 mw1_ref, mb1_ref, mw2_ref, mb2_ref,
                 w1a_ref, w1b_ref, b1_ref,
                 ndst_ref, nsrc_ref):
    h = h_ref[...]
    p = jnp.tanh(jnp.dot(h, pw1_ref[...], preferred_element_type=jnp.float32)
                 + pb1_ref[...])
    p = jnp.dot(p, pw2_ref[...], preferred_element_type=jnp.float32) + pb2_ref[...]
    m = jnp.tanh(jnp.dot(h, mw1_ref[...], preferred_element_type=jnp.float32)
                 + mb1_ref[...])
    m = jnp.dot(m, mw2_ref[...], preferred_element_type=jnp.float32) + mb2_ref[...]
    a = jnp.dot(h, w1a_ref[...], preferred_element_type=jnp.float32) + b1_ref[...]
    b = jnp.dot(h, w1b_ref[...], preferred_element_type=jnp.float32)
    zero = jnp.zeros((h.shape[0], D - H), jnp.float32)
    ndst_ref[:, :H] = a
    ndst_ref[:, H:D] = zero
    ndst_ref[:, D:] = -p
    nsrc_ref[:, :H] = b
    nsrc_ref[:, H:D] = zero
    nsrc_ref[:, D:] = m


def _node_precompute(h, p):
    pw1, pb1, pw2, pb2 = p[0], p[1], p[2], p[3]
    mw1, mb1, mw2, mb2 = p[4], p[5], p[6], p[7]
    w1, b1 = p[8], p[9]
    w1a, w1b = w1[:D], w1[D:]
    full = lambda s: pl.BlockSpec(s, lambda i: (0, 0))
    row = lambda w: pl.BlockSpec((_NB, w), lambda i: (i, 0))
    return pl.pallas_call(
        _node_kernel,
        grid=(N // _NB,),
        in_specs=[row(D),
                  full((D, H)), full((1, H)), full((H, D)), full((1, D)),
                  full((D, H)), full((1, H)), full((H, D)), full((1, D)),
                  full((D, H)), full((D, H)), full((1, H))],
        out_specs=[row(WIDTH), row(WIDTH)],
        out_shape=[jax.ShapeDtypeStruct((N, WIDTH), jnp.float32),
                   jax.ShapeDtypeStruct((N, WIDTH), jnp.float32)],
    )(h, pw1, pb1.reshape(1, H), pw2, pb2.reshape(1, D),
      mw1, mb1.reshape(1, H), mw2, mb2.reshape(1, D),
      w1a, w1b, b1.reshape(1, H))


@functools.lru_cache(maxsize=None)
def _build_sc_gather():
    mesh = plsc.VectorSubcoreMesh(core_axis_name="c", subcore_axis_name="s")

    @functools.partial(
        pl.kernel,
        out_type=(jax.ShapeDtypeStruct((EPAD, WIDTH), jnp.float32),
                  jax.ShapeDtypeStruct((EPAD, WIDTH), jnp.float32)),
        mesh=mesh,
        scratch_types=[pltpu.VMEM((NCHUNK, CH), jnp.int32),
                       pltpu.VMEM((NCHUNK, CH), jnp.int32),
                       pltpu.VMEM((CH, WIDTH), jnp.float32),
                       pltpu.VMEM((CH, WIDTH), jnp.float32),
                       pltpu.SemaphoreType.DMA,
                       pltpu.SemaphoreType.DMA],
    )
    def sc_gather(ndst_hbm, nsrc_hbm, dst3_hbm, src3_hbm, gd_hbm, gs_hbm,
                  idxd_v, idxs_v, rowd_v, rows_v, semd, sems):
        c = lax.axis_index("c")
        s = lax.axis_index("s")
        wid = s * NC + c
        pltpu.sync_copy(dst3_hbm.at[wid], idxd_v)
        pltpu.sync_copy(src3_hbm.at[wid], idxs_v)

        def body(j, _):
            base = wid * PERW + j * CH
            cpd = pltpu.async_copy(ndst_hbm.at[idxd_v.at[j]], rowd_v, semd)
            cps = pltpu.async_copy(nsrc_hbm.at[idxs_v.at[j]], rows_v, sems)
            cpd.wait()
            cps.wait()
            pltpu.sync_copy(rowd_v, gd_hbm.at[pl.ds(base, CH)])
            pltpu.sync_copy(rows_v, gs_hbm.at[pl.ds(base, CH)])
            return 0

        lax.fori_loop(0, NCHUNK, body, 0)

    return sc_gather


def _sc_gather(ndst, nsrc, dst3, src3):
    return _build_sc_gather()(ndst, nsrc, dst3, src3)


_EB = 2048


def _edge_kernel(gd_ref, gs_ref, w2_ref, b2_ref, out_ref):
    z = gd_ref[...] + gs_ref[...]
    hid = jnp.maximum(z[:, :D], 0.0)
    cw = jax.nn.sigmoid(
        jnp.dot(hid, w2_ref[...], preferred_element_type=jnp.float32)
        + b2_ref[...])
    msg = cw * jnp.sin(z[:, D:])
    i = pl.program_id(0)
    rowid = i * _EB + lax.broadcasted_iota(jnp.int32, (_EB, 1), 0)
    out_ref[...] = jnp.where(rowid < E, msg, 0.0)


def _edge_compute(gd, gs, w2p, b2):
    full = lambda s: pl.BlockSpec(s, lambda i: (0, 0))
    return pl.pallas_call(
        _edge_kernel,
        grid=(EPAD // _EB,),
        in_specs=[pl.BlockSpec((_EB, WIDTH), lambda i: (i, 0)),
                  pl.BlockSpec((_EB, WIDTH), lambda i: (i, 0)),
                  full((D, D)), full((1, D))],
        out_specs=pl.BlockSpec((_EB, D), lambda i: (i, 0)),
        out_shape=jax.ShapeDtypeStruct((EPAD, D), jnp.float32),
    )(gd, gs, w2p, b2.reshape(1, D))


@functools.lru_cache(maxsize=None)
def _build_sc_scatter():
    mesh = plsc.VectorSubcoreMesh(core_axis_name="c", subcore_axis_name="s")

    @functools.partial(
        pl.kernel,
        out_type=jax.ShapeDtypeStruct((NC, NACC, D), jnp.float32),
        mesh=mesh,
        scratch_types=[pltpu.VMEM((NCHUNK, CH), jnp.int32),
                       pltpu.VMEM((CH, D), jnp.float32),
                       pltpu.VMEM_SHARED((NACC, D), jnp.float32)],
    )
    def sc_scatter(msgs_hbm, dst3_hbm, zeros_hbm, out_hbm, idx_v, row_v, acc_sh):
        c = lax.axis_index("c")
        s = lax.axis_index("s")
        wid = s * NC + c
        pltpu.sync_copy(zeros_hbm, acc_sh.at[pl.ds(s * RPT, RPT)])
        pltpu.sync_copy(dst3_hbm.at[wid], idx_v)
        plsc.subcore_barrier()

        def body(j, _):
            base = wid * PERW + j * CH
            pltpu.sync_copy(msgs_hbm.at[pl.ds(base, CH)], row_v)
            pltpu.sync_copy(row_v, acc_sh.at[idx_v.at[j]], add=True)
            return 0

        lax.fori_loop(0, NCHUNK, body, 0)
        plsc.subcore_barrier()
        pltpu.sync_copy(acc_sh.at[pl.ds(s * RPT, RPT)],
                        out_hbm.at[c, pl.ds(s * RPT, RPT)])

    return sc_scatter


def _sc_scatter(msgs, dst3, zeros_slab):
    return _build_sc_scatter()(msgs, dst3, zeros_slab)


@functools.lru_cache(maxsize=None)
def _build_sc_count():
    mesh = plsc.VectorSubcoreMesh(core_axis_name="c", subcore_axis_name="s")

    @functools.partial(
        pl.kernel,
        out_type=jax.ShapeDtypeStruct((NC, NACC, D), jnp.float32),
        mesh=mesh,
        scratch_types=[pltpu.VMEM((NCHUNK, CH), jnp.int32),
                       pltpu.VMEM((CH, D), jnp.float32),
                       pltpu.VMEM_SHARED((NACC, D), jnp.float32)],
    )
    def sc_count(dst3_hbm, ones_hbm, zeros_hbm, out_hbm, idx_v, ones_v, acc_sh):
        c = lax.axis_index("c")
        s = lax.axis_index("s")
        wid = s * NC + c
        pltpu.sync_copy(zeros_hbm, acc_sh.at[pl.ds(s * RPT, RPT)])
        pltpu.sync_copy(dst3_hbm.at[wid], idx_v)
        pltpu.sync_copy(ones_hbm, ones_v)
        plsc.subcore_barrier()

        def body(j, _):
            pltpu.sync_copy(ones_v, acc_sh.at[idx_v.at[j]], add=True)
            return 0

        lax.fori_loop(0, NCHUNK, body, 0)
        plsc.subcore_barrier()
        pltpu.sync_copy(acc_sh.at[pl.ds(s * RPT, RPT)],
                        out_hbm.at[c, pl.ds(s * RPT, RPT)])

    return sc_count


def _sc_count(dst3, ones_rows, zeros_slab):
    return _build_sc_count()(dst3, ones_rows, zeros_slab)


def _out_kernel(s0_ref, s1_ref, c0_ref, c1_ref, res_ref,
                w3_ref, b3_ref, w4_ref, b4_ref, g_ref, be_ref,
                out_ref, *, epilogue):
    stot = s0_ref[0] + s1_ref[0]
    cnt = jnp.maximum(c0_ref[0][:, :1] + c1_ref[0][:, :1], 1.0)
    mean = stot / cnt
    o = jnp.maximum(
        jnp.dot(mean, w3_ref[...], preferred_element_type=jnp.float32)
        + b3_ref[...], 0.0)
    o = jnp.dot(o, w4_ref[...], preferred_element_type=jnp.float32) + b4_ref[...]
    if epilogue == "ln_relu":
        mu = jnp.mean(o, axis=-1, keepdims=True)
        var = jnp.mean((o - mu) ** 2, axis=-1, keepdims=True)
        o = (o - mu) * lax.rsqrt(var + 1e-5) * g_ref[...] + be_ref[...]
        o = jnp.maximum(o, 0.0)
    else:
        o = o + res_ref[...]
    out_ref[...] = o


def _aggregate(s2, cnt2, res, p, g, be, epilogue):
    w3, b3, w4, b4 = p[12], p[13], p[14], p[15]
    full = lambda s: pl.BlockSpec(s, lambda i: (0, 0))
    part = lambda k: pl.BlockSpec((1, _NB, D), lambda i, _k=k: (_k, i, 0))
    return pl.pallas_call(
        functools.partial(_out_kernel, epilogue=epilogue),
        grid=(N // _NB,),
        in_specs=[part(0), part(1), part(0), part(1),
                  pl.BlockSpec((_NB, D), lambda i: (i, 0)),
                  full((D, H)), full((1, H)), full((H, D)), full((1, D)),
                  full((1, D)), full((1, D))],
        out_specs=pl.BlockSpec((_NB, D), lambda i: (i, 0)),
        out_shape=jax.ShapeDtypeStruct((N, D), jnp.float32),
    )(s2, s2, cnt2, cnt2, res, w3, b3.reshape(1, H), w4,
      b4.reshape(1, D), g.reshape(1, D), be.reshape(1, D))


def _layer(h, src3, dstg3, dst3, cnt2, zeros_slab, w2p, p, res, g, be,
           epilogue):
    ndst, nsrc = _node_precompute(h, p)
    gd, gs = _sc_gather(ndst, nsrc, dstg3, src3)
    msgs = _edge_compute(gd, gs, w2p, p[11])
    s2 = _sc_scatter(msgs, dst3, zeros_slab)
    return _aggregate(s2, cnt2, res, p, g, be, epilogue)


@jax.jit
def kernel(x, params, edge_index):
    p0 = params[0:16]
    p1 = params[16:32]
    g, be = params[32], params[33]
    src = edge_index[0]
    dst = edge_index[1]
    zpad = jnp.zeros((EPAD - E,), jnp.int32)
    src3 = jnp.concatenate([src, zpad]).reshape(NW, NCHUNK, CH)
    dstg3 = jnp.concatenate([dst, zpad]).reshape(NW, NCHUNK, CH)
    dst3 = jnp.concatenate(
        [dst, jnp.full((EPAD - E,), N, jnp.int32)]).reshape(NW, NCHUNK, CH)
    zeros_slab = jnp.zeros((RPT, D), jnp.float32)
    ones_rows = jnp.ones((CH, D), jnp.float32)
    w2p0 = jnp.concatenate([p0[10], jnp.zeros((D - H, D), jnp.float32)])
    w2p1 = jnp.concatenate([p1[10], jnp.zeros((D - H, D), jnp.float32)])
    cnt2 = _sc_count(dst3, ones_rows, zeros_slab)
    h = _layer(x, src3, dstg3, dst3, cnt2, zeros_slab, w2p0, p0, x, g, be,
               "ln_relu")
    out = _layer(h, src3, dstg3, dst3, cnt2, zeros_slab, w2p1, p1, x, g, be,
                 "residual")
    return out


# speedup vs baseline: 2.0517x; 1.2294x over previous
import functools

import jax
import jax.numpy as jnp
from jax import lax
from jax.experimental import pallas as pl
from jax.experimental.pallas import tpu as pltpu
from jax.experimental.pallas import tpu_sc as plsc

N = 10000
E = 320000
D = 128
H = 64

NC = 2
NS = 16
NW = NC * NS
CH = 128
NCHUNK = 80
PERW = CH * NCHUNK
EPAD = PERW * NW
WIDTH = 2 * D
NACC = 10112
RPT = NACC // NS


_NB = 1000


def _node_kernel(h_ref, pw1_ref, pb1_ref, pw2_ref, pb2_ref,
                 mw1_ref, mb1_ref, mw2_ref, mb2_ref,
                 w1a_ref, w1b_ref, b1_ref,
                 ndst_ref, nsrc_ref):
    h = h_ref[...]
    p = jnp.tanh(jnp.dot(h, pw1_ref[...], preferred_element_type=jnp.float32)
                 + pb1_ref[...])
    p = jnp.dot(p, pw2_ref[...], preferred_element_type=jnp.float32) + pb2_ref[...]
    m = jnp.tanh(jnp.dot(h, mw1_ref[...], preferred_element_type=jnp.float32)
                 + mb1_ref[...])
    m = jnp.dot(m, mw2_ref[...], preferred_element_type=jnp.float32) + mb2_ref[...]
    a = jnp.dot(h, w1a_ref[...], preferred_element_type=jnp.float32) + b1_ref[...]
    b = jnp.dot(h, w1b_ref[...], preferred_element_type=jnp.float32)
    zero = jnp.zeros((h.shape[0], D - H), jnp.float32)
    ndst_ref[:, :H] = a
    ndst_ref[:, H:D] = zero
    ndst_ref[:, D:] = -p
    nsrc_ref[:, :H] = b
    nsrc_ref[:, H:D] = zero
    nsrc_ref[:, D:] = m


def _node_precompute(h, p):
    pw1, pb1, pw2, pb2 = p[0], p[1], p[2], p[3]
    mw1, mb1, mw2, mb2 = p[4], p[5], p[6], p[7]
    w1, b1 = p[8], p[9]
    w1a, w1b = w1[:D], w1[D:]
    full = lambda s: pl.BlockSpec(s, lambda i: (0, 0))
    row = lambda w: pl.BlockSpec((_NB, w), lambda i: (i, 0))
    return pl.pallas_call(
        _node_kernel,
        grid=(N // _NB,),
        in_specs=[row(D),
                  full((D, H)), full((1, H)), full((H, D)), full((1, D)),
                  full((D, H)), full((1, H)), full((H, D)), full((1, D)),
                  full((D, H)), full((D, H)), full((1, H))],
        out_specs=[row(WIDTH), row(WIDTH)],
        out_shape=[jax.ShapeDtypeStruct((N, WIDTH), jnp.float32),
                   jax.ShapeDtypeStruct((N, WIDTH), jnp.float32)],
    )(h, pw1, pb1.reshape(1, H), pw2, pb2.reshape(1, D),
      mw1, mb1.reshape(1, H), mw2, mb2.reshape(1, D),
      w1a, w1b, b1.reshape(1, H))


GCH = 64
NGCH = PERW // GCH


@functools.lru_cache(maxsize=None)
def _build_sc_gather():
    mesh = plsc.VectorSubcoreMesh(core_axis_name="c", subcore_axis_name="s")

    @functools.partial(
        pl.kernel,
        out_type=(jax.ShapeDtypeStruct((EPAD, WIDTH), jnp.float32),
                  jax.ShapeDtypeStruct((EPAD, WIDTH), jnp.float32)),
        mesh=mesh,
        scratch_types=[pltpu.VMEM((NGCH, GCH), jnp.int32),
                       pltpu.VMEM((NGCH, GCH), jnp.int32),
                       pltpu.VMEM((GCH, WIDTH), jnp.float32),
                       pltpu.VMEM((GCH, WIDTH), jnp.float32),
                       pltpu.VMEM((GCH, WIDTH), jnp.float32),
                       pltpu.VMEM((GCH, WIDTH), jnp.float32),
                       pltpu.SemaphoreType.DMA, pltpu.SemaphoreType.DMA,
                       pltpu.SemaphoreType.DMA, pltpu.SemaphoreType.DMA,
                       pltpu.SemaphoreType.DMA, pltpu.SemaphoreType.DMA,
                       pltpu.SemaphoreType.DMA, pltpu.SemaphoreType.DMA],
    )
    def sc_gather(ndst_hbm, nsrc_hbm, dst3_hbm, src3_hbm, gd_hbm, gs_hbm,
                  idxd_v, idxs_v, bd0, bs0, bd1, bs1,
                  gd0, gs0, gd1, gs1, wd0, ws0, wd1, ws1):
        c = lax.axis_index("c")
        s = lax.axis_index("s")
        wid = s * NC + c
        pltpu.sync_copy(dst3_hbm.at[wid], idxd_v)
        pltpu.sync_copy(src3_hbm.at[wid], idxs_v)
        bufs = ((bd0, bs0), (bd1, bs1))
        gsem = ((gd0, gs0), (gd1, gs1))
        wsem = ((wd0, ws0), (wd1, ws1))

        def g_start(g, b):
            pltpu.async_copy(ndst_hbm.at[idxd_v.at[g]], bufs[b][0], gsem[b][0])
            pltpu.async_copy(nsrc_hbm.at[idxs_v.at[g]], bufs[b][1], gsem[b][1])

        def g_wait(g, b):
            pltpu.make_async_copy(ndst_hbm.at[idxd_v.at[g]], bufs[b][0],
                                  gsem[b][0]).wait()
            pltpu.make_async_copy(nsrc_hbm.at[idxs_v.at[g]], bufs[b][1],
                                  gsem[b][1]).wait()

        def w_start(g, b):
            base = wid * PERW + g * GCH
            pltpu.async_copy(bufs[b][0], gd_hbm.at[pl.ds(base, GCH)], wsem[b][0])
            pltpu.async_copy(bufs[b][1], gs_hbm.at[pl.ds(base, GCH)], wsem[b][1])

        def w_wait(g, b):
            base = wid * PERW + g * GCH
            pltpu.make_async_copy(bufs[b][0], gd_hbm.at[pl.ds(base, GCH)],
                                  wsem[b][0]).wait()
            pltpu.make_async_copy(bufs[b][1], gs_hbm.at[pl.ds(base, GCH)],
                                  wsem[b][1]).wait()

        g_start(0, 0)
        g_start(1, 1)

        def body(i, _):
            for b in (0, 1):
                g = 2 * i + b
                g_wait(g, b)
                w_start(g, b)
                w_wait(g, b)
                g_start(g + 2, b)
            return 0

        lax.fori_loop(0, NGCH // 2 - 1, body, 0)
        for b in (0, 1):
            g = NGCH - 2 + b
            g_wait(g, b)
            w_start(g, b)
            w_wait(g, b)

    return sc_gather


def _sc_gather(ndst, nsrc, dst3, src3):
    return _build_sc_gather()(ndst, nsrc, dst3, src3)


_EB = 2048


def _edge_kernel(gd_ref, gs_ref, w2_ref, b2_ref, out_ref):
    z = gd_ref[...] + gs_ref[...]
    hid = jnp.maximum(z[:, :D], 0.0)
    cw = jax.nn.sigmoid(
        jnp.dot(hid, w2_ref[...], preferred_element_type=jnp.float32)
        + b2_ref[...])
    msg = cw * jnp.sin(z[:, D:])
    i = pl.program_id(0)
    rowid = i * _EB + lax.broadcasted_iota(jnp.int32, (_EB, 1), 0)
    out_ref[...] = jnp.where(rowid < E, msg, 0.0)


def _edge_compute(gd, gs, w2p, b2):
    full = lambda s: pl.BlockSpec(s, lambda i: (0, 0))
    return pl.pallas_call(
        _edge_kernel,
        grid=(EPAD // _EB,),
        in_specs=[pl.BlockSpec((_EB, WIDTH), lambda i: (i, 0)),
                  pl.BlockSpec((_EB, WIDTH), lambda i: (i, 0)),
                  full((D, D)), full((1, D))],
        out_specs=pl.BlockSpec((_EB, D), lambda i: (i, 0)),
        out_shape=jax.ShapeDtypeStruct((EPAD, D), jnp.float32),
    )(gd, gs, w2p, b2.reshape(1, D))


@functools.lru_cache(maxsize=None)
def _build_sc_scatter():
    mesh = plsc.VectorSubcoreMesh(core_axis_name="c", subcore_axis_name="s")

    @functools.partial(
        pl.kernel,
        out_type=jax.ShapeDtypeStruct((NC, NACC, D), jnp.float32),
        mesh=mesh,
        scratch_types=[pltpu.VMEM((NCHUNK, CH), jnp.int32),
                       pltpu.VMEM((CH, D), jnp.float32),
                       pltpu.VMEM_SHARED((NACC, D), jnp.float32)],
    )
    def sc_scatter(msgs_hbm, dst3_hbm, zeros_hbm, out_hbm, idx_v, row_v, acc_sh):
        c = lax.axis_index("c")
        s = lax.axis_index("s")
        wid = s * NC + c
        pltpu.sync_copy(zeros_hbm, acc_sh.at[pl.ds(s * RPT, RPT)])
        pltpu.sync_copy(dst3_hbm.at[wid], idx_v)
        plsc.subcore_barrier()

        def body(j, _):
            base = wid * PERW + j * CH
            pltpu.sync_copy(msgs_hbm.at[pl.ds(base, CH)], row_v)
            pltpu.sync_copy(row_v, acc_sh.at[idx_v.at[j]], add=True)
            return 0

        lax.fori_loop(0, NCHUNK, body, 0)
        plsc.subcore_barrier()
        pltpu.sync_copy(acc_sh.at[pl.ds(s * RPT, RPT)],
                        out_hbm.at[c, pl.ds(s * RPT, RPT)])

    return sc_scatter


def _sc_scatter(msgs, dst3, zeros_slab):
    return _build_sc_scatter()(msgs, dst3, zeros_slab)


@functools.lru_cache(maxsize=None)
def _build_sc_count():
    mesh = plsc.VectorSubcoreMesh(core_axis_name="c", subcore_axis_name="s")

    @functools.partial(
        pl.kernel,
        out_type=jax.ShapeDtypeStruct((NC, NACC, D), jnp.float32),
        mesh=mesh,
        scratch_types=[pltpu.VMEM((NCHUNK, CH), jnp.int32),
                       pltpu.VMEM((CH, D), jnp.float32),
                       pltpu.VMEM_SHARED((NACC, D), jnp.float32)],
    )
    def sc_count(dst3_hbm, ones_hbm, zeros_hbm, out_hbm, idx_v, ones_v, acc_sh):
        c = lax.axis_index("c")
        s = lax.axis_index("s")
        wid = s * NC + c
        pltpu.sync_copy(zeros_hbm, acc_sh.at[pl.ds(s * RPT, RPT)])
        pltpu.sync_copy(dst3_hbm.at[wid], idx_v)
        pltpu.sync_copy(ones_hbm, ones_v)
        plsc.subcore_barrier()

        def body(j, _):
            pltpu.sync_copy(ones_v, acc_sh.at[idx_v.at[j]], add=True)
            return 0

        lax.fori_loop(0, NCHUNK, body, 0)
        plsc.subcore_barrier()
        pltpu.sync_copy(acc_sh.at[pl.ds(s * RPT, RPT)],
                        out_hbm.at[c, pl.ds(s * RPT, RPT)])

    return sc_count


def _sc_count(dst3, ones_rows, zeros_slab):
    return _build_sc_count()(dst3, ones_rows, zeros_slab)


def _out_kernel(s0_ref, s1_ref, c0_ref, c1_ref, res_ref,
                w3_ref, b3_ref, w4_ref, b4_ref, g_ref, be_ref,
                out_ref, *, epilogue):
    stot = s0_ref[0] + s1_ref[0]
    cnt = jnp.maximum(c0_ref[0][:, :1] + c1_ref[0][:, :1], 1.0)
    mean = stot / cnt
    o = jnp.maximum(
        jnp.dot(mean, w3_ref[...], preferred_element_type=jnp.float32)
        + b3_ref[...], 0.0)
    o = jnp.dot(o, w4_ref[...], preferred_element_type=jnp.float32) + b4_ref[...]
    if epilogue == "ln_relu":
        mu = jnp.mean(o, axis=-1, keepdims=True)
        var = jnp.mean((o - mu) ** 2, axis=-1, keepdims=True)
        o = (o - mu) * lax.rsqrt(var + 1e-5) * g_ref[...] + be_ref[...]
        o = jnp.maximum(o, 0.0)
    else:
        o = o + res_ref[...]
    out_ref[...] = o


def _aggregate(s2, cnt2, res, p, g, be, epilogue):
    w3, b3, w4, b4 = p[12], p[13], p[14], p[15]
    full = lambda s: pl.BlockSpec(s, lambda i: (0, 0))
    part = lambda k: pl.BlockSpec((1, _NB, D), lambda i, _k=k: (_k, i, 0))
    return pl.pallas_call(
        functools.partial(_out_kernel, epilogue=epilogue),
        grid=(N // _NB,),
        in_specs=[part(0), part(1), part(0), part(1),
                  pl.BlockSpec((_NB, D), lambda i: (i, 0)),
                  full((D, H)), full((1, H)), full((H, D)), full((1, D)),
                  full((1, D)), full((1, D))],
        out_specs=pl.BlockSpec((_NB, D), lambda i: (i, 0)),
        out_shape=jax.ShapeDtypeStruct((N, D), jnp.float32),
    )(s2, s2, cnt2, cnt2, res, w3, b3.reshape(1, H), w4,
      b4.reshape(1, D), g.reshape(1, D), be.reshape(1, D))


def _layer(h, src3, dstg3, dst3, cnt2, zeros_slab, w2p, p, res, g, be,
           epilogue):
    ndst, nsrc = _node_precompute(h, p)
    gd, gs = _sc_gather(ndst, nsrc, dstg3, src3)
    msgs = _edge_compute(gd, gs, w2p, p[11])
    s2 = _sc_scatter(msgs, dst3, zeros_slab)
    return _aggregate(s2, cnt2, res, p, g, be, epilogue)


@jax.jit
def kernel(x, params, edge_index):
    p0 = params[0:16]
    p1 = params[16:32]
    g, be = params[32], params[33]
    src = edge_index[0]
    dst = edge_index[1]
    zpad = jnp.zeros((EPAD - E,), jnp.int32)
    src3 = jnp.concatenate([src, zpad]).reshape(NW, NGCH, GCH)
    dstg3 = jnp.concatenate([dst, zpad]).reshape(NW, NGCH, GCH)
    dst3 = jnp.concatenate(
        [dst, jnp.full((EPAD - E,), N, jnp.int32)]).reshape(NW, NCHUNK, CH)
    zeros_slab = jnp.zeros((RPT, D), jnp.float32)
    ones_rows = jnp.ones((CH, D), jnp.float32)
    w2p0 = jnp.concatenate([p0[10], jnp.zeros((D - H, D), jnp.float32)])
    w2p1 = jnp.concatenate([p1[10], jnp.zeros((D - H, D), jnp.float32)])
    cnt2 = _sc_count(dst3, ones_rows, zeros_slab)
    h = _layer(x, src3, dstg3, dst3, cnt2, zeros_slab, w2p0, p0, x, g, be,
               "ln_relu")
    out = _layer(h, src3, dstg3, dst3, cnt2, zeros_slab, w2p1, p1, x, g, be,
                 "residual")
    return out


# speedup vs baseline: 2.2289x; 1.0864x over previous
import functools

import jax
import jax.numpy as jnp
from jax import lax
from jax.experimental import pallas as pl
from jax.experimental.pallas import tpu as pltpu
from jax.experimental.pallas import tpu_sc as plsc

N = 10000
E = 320000
D = 128
H = 64

NC = 2
NS = 16
NW = NC * NS
CH = 128
NCHUNK = 80
PERW = CH * NCHUNK
EPAD = PERW * NW
WIDTH = 2 * D
NACC = 10112
RPT = NACC // NS


_NB = 1000


def _node_kernel(h_ref, pw1_ref, pb1_ref, pw2_ref, pb2_ref,
                 mw1_ref, mb1_ref, mw2_ref, mb2_ref,
                 w1a_ref, w1b_ref, b1_ref,
                 ndst_ref, nsrc_ref):
    h = h_ref[...]
    p = jnp.tanh(jnp.dot(h, pw1_ref[...], preferred_element_type=jnp.float32)
                 + pb1_ref[...])
    p = jnp.dot(p, pw2_ref[...], preferred_element_type=jnp.float32) + pb2_ref[...]
    m = jnp.tanh(jnp.dot(h, mw1_ref[...], preferred_element_type=jnp.float32)
                 + mb1_ref[...])
    m = jnp.dot(m, mw2_ref[...], preferred_element_type=jnp.float32) + mb2_ref[...]
    a = jnp.dot(h, w1a_ref[...], preferred_element_type=jnp.float32) + b1_ref[...]
    b = jnp.dot(h, w1b_ref[...], preferred_element_type=jnp.float32)
    zero = jnp.zeros((h.shape[0], D - H), jnp.float32)
    ndst_ref[:, :H] = a
    ndst_ref[:, H:D] = zero
    ndst_ref[:, D:] = -p
    nsrc_ref[:, :H] = b
    nsrc_ref[:, H:D] = zero
    nsrc_ref[:, D:] = m


def _node_precompute(h, p):
    pw1, pb1, pw2, pb2 = p[0], p[1], p[2], p[3]
    mw1, mb1, mw2, mb2 = p[4], p[5], p[6], p[7]
    w1, b1 = p[8], p[9]
    w1a, w1b = w1[:D], w1[D:]
    full = lambda s: pl.BlockSpec(s, lambda i: (0, 0))
    row = lambda w: pl.BlockSpec((_NB, w), lambda i: (i, 0))
    return pl.pallas_call(
        _node_kernel,
        grid=(N // _NB,),
        in_specs=[row(D),
                  full((D, H)), full((1, H)), full((H, D)), full((1, D)),
                  full((D, H)), full((1, H)), full((H, D)), full((1, D)),
                  full((D, H)), full((D, H)), full((1, H))],
        out_specs=[row(WIDTH), row(WIDTH)],
        out_shape=[jax.ShapeDtypeStruct((N, WIDTH), jnp.float32),
                   jax.ShapeDtypeStruct((N, WIDTH), jnp.float32)],
    )(h, pw1, pb1.reshape(1, H), pw2, pb2.reshape(1, D),
      mw1, mb1.reshape(1, H), mw2, mb2.reshape(1, D),
      w1a, w1b, b1.reshape(1, H))


GCH = 64
CPS = 320
NG0 = 208
NG1 = CPS - NG0
IDXR = NG0 // 2


@functools.lru_cache(maxsize=None)
def _build_sc_gather():
    mesh = plsc.VectorSubcoreMesh(core_axis_name="c", subcore_axis_name="s")

    @functools.partial(
        pl.kernel,
        out_type=jax.ShapeDtypeStruct((EPAD, WIDTH), jnp.float32),
        mesh=mesh,
        scratch_types=[pltpu.VMEM((IDXR, 2 * GCH), jnp.int32),
                       pltpu.VMEM((IDXR, 2 * GCH), jnp.int32),
                       pltpu.VMEM((GCH, WIDTH), jnp.float32),
                       pltpu.VMEM((GCH, WIDTH), jnp.float32),
                       pltpu.VMEM((GCH, WIDTH), jnp.float32),
                       pltpu.VMEM((GCH, WIDTH), jnp.float32),
                       pltpu.SemaphoreType.DMA, pltpu.SemaphoreType.DMA,
                       pltpu.SemaphoreType.DMA, pltpu.SemaphoreType.DMA,
                       pltpu.SemaphoreType.DMA, pltpu.SemaphoreType.DMA],
    )
    def sc_gather(ndst_hbm, nsrc_hbm, dst4_hbm, src4_hbm, gz_hbm,
                  idxd_v, idxs_v, bd0, bs0, bd1, bs1,
                  gd0, gs0, gd1, gs1, w0, w1):
        c = lax.axis_index("c")
        s = lax.axis_index("s")
        wid = s * NC + c
        pltpu.sync_copy(dst4_hbm.at[wid], idxd_v)
        pltpu.sync_copy(src4_hbm.at[wid], idxs_v)
        bufs = ((bd0, bs0), (bd1, bs1))
        gsem = ((gd0, gs0), (gd1, gs1))
        wsem = (w0, w1)
        npairs = jnp.where(c == 0, NG0 // 2, NG1 // 2)
        cbase = s * CPS + c * NG0

        def g_start(i, b):
            idxd = idxd_v.at[i, pl.ds(64 * b, GCH)]
            idxs = idxs_v.at[i, pl.ds(64 * b, GCH)]
            pltpu.async_copy(ndst_hbm.at[idxd], bufs[b][0], gsem[b][0])
            pltpu.async_copy(nsrc_hbm.at[idxs], bufs[b][1], gsem[b][1])

        def g_wait(i, b):
            idxd = idxd_v.at[i, pl.ds(64 * b, GCH)]
            idxs = idxs_v.at[i, pl.ds(64 * b, GCH)]
            pltpu.make_async_copy(ndst_hbm.at[idxd], bufs[b][0],
                                  gsem[b][0]).wait()
            pltpu.make_async_copy(nsrc_hbm.at[idxs], bufs[b][1],
                                  gsem[b][1]).wait()

        def combine(b):
            bd, bs = bufs[b]

            def crow(r, _):
                for k in range(4):
                    sl = pl.ds(16 * k, 16)
                    bd[r, sl] = bd[r, sl] + bs[r, sl]
                for k in range(8):
                    sl = pl.ds(D + 16 * k, 16)
                    bd[r, sl] = bd[r, sl] + bs[r, sl]
                return 0

            lax.fori_loop(0, GCH, crow, 0)

        def w_start(i, b):
            base = (cbase + 2 * i + b) * GCH
            pltpu.async_copy(bufs[b][0], gz_hbm.at[pl.ds(base, GCH)], wsem[b])

        def w_wait(i, b):
            base = (cbase + 2 * i + b) * GCH
            pltpu.make_async_copy(bufs[b][0], gz_hbm.at[pl.ds(base, GCH)],
                                  wsem[b]).wait()

        g_start(0, 0)
        g_start(0, 1)

        def body(i, _):
            for b in (0, 1):
                g_wait(i, b)
                combine(b)
                w_start(i, b)
                w_wait(i, b)
                g_start(i + 1, b)
            return 0

        lax.fori_loop(0, npairs - 1, body, 0)
        i_last = npairs - 1
        for b in (0, 1):
            g_wait(i_last, b)
            combine(b)
            w_start(i_last, b)
            w_wait(i_last, b)

    return sc_gather


def _sc_gather(ndst, nsrc, dst4, src4):
    return _build_sc_gather()(ndst, nsrc, dst4, src4)


_EB = 2048


def _edge_kernel(gz_ref, w2_ref, b2_ref, out_ref):
    z = gz_ref[...]
    hid = jnp.maximum(z[:, :D], 0.0)
    cw = jax.nn.sigmoid(
        jnp.dot(hid, w2_ref[...], preferred_element_type=jnp.float32)
        + b2_ref[...])
    msg = cw * jnp.sin(z[:, D:])
    i = pl.program_id(0)
    rowid = i * _EB + lax.broadcasted_iota(jnp.int32, (_EB, 1), 0)
    out_ref[...] = jnp.where(rowid < E, msg, 0.0)


def _edge_compute(gz, w2p, b2):
    full = lambda s: pl.BlockSpec(s, lambda i: (0, 0))
    return pl.pallas_call(
        _edge_kernel,
        grid=(EPAD // _EB,),
        in_specs=[pl.BlockSpec((_EB, WIDTH), lambda i: (i, 0)),
                  full((D, D)), full((1, D))],
        out_specs=pl.BlockSpec((_EB, D), lambda i: (i, 0)),
        out_shape=jax.ShapeDtypeStruct((EPAD, D), jnp.float32),
    )(gz, w2p, b2.reshape(1, D))


@functools.lru_cache(maxsize=None)
def _build_sc_scatter():
    mesh = plsc.VectorSubcoreMesh(core_axis_name="c", subcore_axis_name="s")

    @functools.partial(
        pl.kernel,
        out_type=jax.ShapeDtypeStruct((NC, NACC, D), jnp.float32),
        mesh=mesh,
        scratch_types=[pltpu.VMEM((NCHUNK, CH), jnp.int32),
                       pltpu.VMEM((CH, D), jnp.float32),
                       pltpu.VMEM_SHARED((NACC, D), jnp.float32)],
    )
    def sc_scatter(msgs_hbm, dst3_hbm, zeros_hbm, out_hbm, idx_v, row_v, acc_sh):
        c = lax.axis_index("c")
        s = lax.axis_index("s")
        wid = s * NC + c
        pltpu.sync_copy(zeros_hbm, acc_sh.at[pl.ds(s * RPT, RPT)])
        pltpu.sync_copy(dst3_hbm.at[wid], idx_v)
        plsc.subcore_barrier()

        def body(j, _):
            base = wid * PERW + j * CH
            pltpu.sync_copy(msgs_hbm.at[pl.ds(base, CH)], row_v)
            pltpu.sync_copy(row_v, acc_sh.at[idx_v.at[j]], add=True)
            return 0

        lax.fori_loop(0, NCHUNK, body, 0)
        plsc.subcore_barrier()
        pltpu.sync_copy(acc_sh.at[pl.ds(s * RPT, RPT)],
                        out_hbm.at[c, pl.ds(s * RPT, RPT)])

    return sc_scatter


def _sc_scatter(msgs, dst3, zeros_slab):
    return _build_sc_scatter()(msgs, dst3, zeros_slab)


@functools.lru_cache(maxsize=None)
def _build_sc_count():
    mesh = plsc.VectorSubcoreMesh(core_axis_name="c", subcore_axis_name="s")

    @functools.partial(
        pl.kernel,
        out_type=jax.ShapeDtypeStruct((NC, NACC, D), jnp.float32),
        mesh=mesh,
        scratch_types=[pltpu.VMEM((NCHUNK, CH), jnp.int32),
                       pltpu.VMEM((CH, D), jnp.float32),
                       pltpu.VMEM_SHARED((NACC, D), jnp.float32)],
    )
    def sc_count(dst3_hbm, ones_hbm, zeros_hbm, out_hbm, idx_v, ones_v, acc_sh):
        c = lax.axis_index("c")
        s = lax.axis_index("s")
        wid = s * NC + c
        pltpu.sync_copy(zeros_hbm, acc_sh.at[pl.ds(s * RPT, RPT)])
        pltpu.sync_copy(dst3_hbm.at[wid], idx_v)
        pltpu.sync_copy(ones_hbm, ones_v)
        plsc.subcore_barrier()

        def body(j, _):
            pltpu.sync_copy(ones_v, acc_sh.at[idx_v.at[j]], add=True)
            return 0

        lax.fori_loop(0, NCHUNK, body, 0)
        plsc.subcore_barrier()
        pltpu.sync_copy(acc_sh.at[pl.ds(s * RPT, RPT)],
                        out_hbm.at[c, pl.ds(s * RPT, RPT)])

    return sc_count


def _sc_count(dst3, ones_rows, zeros_slab):
    return _build_sc_count()(dst3, ones_rows, zeros_slab)


def _out_kernel(s0_ref, s1_ref, c0_ref, c1_ref, res_ref,
                w3_ref, b3_ref, w4_ref, b4_ref, g_ref, be_ref,
                out_ref, *, epilogue):
    stot = s0_ref[0] + s1_ref[0]
    cnt = jnp.maximum(c0_ref[0][:, :1] + c1_ref[0][:, :1], 1.0)
    mean = stot / cnt
    o = jnp.maximum(
        jnp.dot(mean, w3_ref[...], preferred_element_type=jnp.float32)
        + b3_ref[...], 0.0)
    o = jnp.dot(o, w4_ref[...], preferred_element_type=jnp.float32) + b4_ref[...]
    if epilogue == "ln_relu":
        mu = jnp.mean(o, axis=-1, keepdims=True)
        var = jnp.mean((o - mu) ** 2, axis=-1, keepdims=True)
        o = (o - mu) * lax.rsqrt(var + 1e-5) * g_ref[...] + be_ref[...]
        o = jnp.maximum(o, 0.0)
    else:
        o = o + res_ref[...]
    out_ref[...] = o


def _aggregate(s2, cnt2, res, p, g, be, epilogue):
    w3, b3, w4, b4 = p[12], p[13], p[14], p[15]
    full = lambda s: pl.BlockSpec(s, lambda i: (0, 0))
    part = lambda k: pl.BlockSpec((1, _NB, D), lambda i, _k=k: (_k, i, 0))
    return pl.pallas_call(
        functools.partial(_out_kernel, epilogue=epilogue),
        grid=(N // _NB,),
        in_specs=[part(0), part(1), part(0), part(1),
                  pl.BlockSpec((_NB, D), lambda i: (i, 0)),
                  full((D, H)), full((1, H)), full((H, D)), full((1, D)),
                  full((1, D)), full((1, D))],
        out_specs=pl.BlockSpec((_NB, D), lambda i: (i, 0)),
        out_shape=jax.ShapeDtypeStruct((N, D), jnp.float32),
    )(s2, s2, cnt2, cnt2, res, w3, b3.reshape(1, H), w4,
      b4.reshape(1, D), g.reshape(1, D), be.reshape(1, D))


def _layer(h, src4, dstg4, dst3, cnt2, zeros_slab, w2p, p, res, g, be,
           epilogue):
    ndst, nsrc = _node_precompute(h, p)
    gz = _sc_gather(ndst, nsrc, dstg4, src4)
    msgs = _edge_compute(gz, w2p, p[11])
    s2 = _sc_scatter(msgs, dst3, zeros_slab)
    return _aggregate(s2, cnt2, res, p, g, be, epilogue)


@jax.jit
def kernel(x, params, edge_index):
    p0 = params[0:16]
    p1 = params[16:32]
    g, be = params[32], params[33]
    src = edge_index[0]
    dst = edge_index[1]
    zpad = jnp.zeros((EPAD - E,), jnp.int32)

    def tile_idx(flat):
        a = flat.reshape(NS, CPS, GCH)
        a0 = a[:, :NG0].reshape(NS, 1, IDXR, 2 * GCH)
        a1 = jnp.concatenate(
            [a[:, NG0:].reshape(NS, NG1 // 2, 2 * GCH),
             jnp.zeros((NS, IDXR - NG1 // 2, 2 * GCH), jnp.int32)],
            axis=1).reshape(NS, 1, IDXR, 2 * GCH)
        return jnp.concatenate([a0, a1], axis=1).reshape(NW, IDXR, 2 * GCH)

    src4 = tile_idx(jnp.concatenate([src, zpad]))
    dstg4 = tile_idx(jnp.concatenate([dst, zpad]))
    dst3 = jnp.concatenate(
        [dst, jnp.full((EPAD - E,), N, jnp.int32)]).reshape(NW, NCHUNK, CH)
    zeros_slab = jnp.zeros((RPT, D), jnp.float32)
    ones_rows = jnp.ones((CH, D), jnp.float32)
    w2p0 = jnp.concatenate([p0[10], jnp.zeros((D - H, D), jnp.float32)])
    w2p1 = jnp.concatenate([p1[10], jnp.zeros((D - H, D), jnp.float32)])
    cnt2 = _sc_count(dst3, ones_rows, zeros_slab)
    h = _layer(x, src4, dstg4, dst3, cnt2, zeros_slab, w2p0, p0, x, g, be,
               "ln_relu")
    out = _layer(h, src4, dstg4, dst3, cnt2, zeros_slab, w2p1, p1, x, g, be,
                 "residual")
    return out
